# baseline (speedup 1.0000x reference)
import functools

import numpy as np
import jax
import jax.numpy as jnp
from jax import lax
from jax.experimental import pallas as pl
from jax.experimental.pallas import tpu as pltpu
from jax.experimental.pallas import tpu_sc as plsc

_NC, _NS, _L = 2, 16, 16
_NW = _NC * _NS
_P = 0.15
_NSTREAM = 4


def _threefry2x32(k0, k1, x0, x1):
    x0 = x0.astype(np.uint32).copy()
    x1 = x1.astype(np.uint32).copy()

    def rotl(v, r):
        return ((v << np.uint32(r)) | (v >> np.uint32(32 - r))).astype(np.uint32)

    ks0 = np.uint32(k0)
    ks1 = np.uint32(k1)
    ks2 = np.uint32(ks0 ^ ks1 ^ np.uint32(0x1BD11BDA))
    ks = (ks0, ks1, ks2)
    x0 = (x0 + ks0).astype(np.uint32)
    x1 = (x1 + ks1).astype(np.uint32)
    r1 = (13, 15, 26, 6)
    r2 = (17, 29, 16, 24)
    for r in range(5):
        for rot in (r1 if r % 2 == 0 else r2):
            x0 = (x0 + x1).astype(np.uint32)
            x1 = rotl(x1, rot) ^ x0
        x0 = (x0 + ks[(r + 1) % 3]).astype(np.uint32)
        x1 = (x1 + ks[(r + 2) % 3] + np.uint32(r + 1)).astype(np.uint32)
    return x0, x1


def _np_bits(k0, k1, n):
    y0, y1 = _threefry2x32(k0, k1, np.zeros(n, np.uint32),
                           np.arange(n, dtype=np.uint32))
    return y0 ^ y1


def _np_split(k0, k1):
    y0, y1 = _threefry2x32(k0, k1, np.zeros(2, np.uint32),
                           np.arange(2, dtype=np.uint32))
    return (y0[0], y1[0]), (y0[1], y1[1])


def _np_uniform(k0, k1, m):
    bits = _np_bits(k0, k1, m)
    fb = (bits >> np.uint32(9)) | np.uint32(0x3F800000)
    return fb.view(np.float32) - np.float32(1.0)


@functools.lru_cache(maxsize=None)
def _swap_plan(b, c):
    n = b * c
    (k10, k11), (k20, k21) = _np_split(0, 42)
    u1 = _np_uniform(k10, k11, n).reshape(b, c)
    u2 = _np_uniform(k20, k21, n).reshape(b, c)
    mask = u1 > np.float32(1.0 - _P)
    rows = np.floor(u2 * np.float32(b)).astype(np.int32)
    delta = (rows.astype(np.int64) * mask.astype(np.int64) * c).reshape(-1)
    src = np.arange(n, dtype=np.int64) + delta
    src = np.where(src >= n, src - n, src).astype(np.int32)

    rows_per = b // _NW
    cs = rows_per * c
    swapped = np.nonzero(src != np.arange(n, dtype=np.int32))[0]
    per = [swapped[(swapped >= w * cs) & (swapped < (w + 1) * cs)]
           for w in range(_NW)]
    kmax = max(len(p) for p in per)
    kpad = -(-kmax // (_L * _NSTREAM)) * (_L * _NSTREAM)
    src_all = np.zeros((_NW, kpad), dtype=np.int32)
    offr_all = np.empty((_NW, kpad), dtype=np.int32)
    offc_all = np.empty((_NW, kpad), dtype=np.int32)
    offr_all[:] = rows_per
    offc_all[:] = np.arange(kpad, dtype=np.int32) % _L
    for w, p in enumerate(per):
        loc = (p - w * cs).astype(np.int32)
        order = np.argsort(src[p], kind="stable")
        src_all[w, :len(p)] = src[p][order]
        offr_all[w, :len(p)] = (loc // c)[order]
        offc_all[w, :len(p)] = (loc % c)[order]
    return rows_per, kpad, src_all, offr_all, offc_all


@functools.lru_cache(maxsize=None)
def _build(b, c):
    rows_per, kpad, src_all, offr_all, offc_all = _swap_plan(b, c)
    mesh = plsc.VectorSubcoreMesh(core_axis_name="c", subcore_axis_name="s",
                                  num_cores=_NC, num_subcores=_NS)

    @functools.partial(
        pl.kernel,
        out_type=jax.ShapeDtypeStruct((b, c), jnp.float32),
        mesh=mesh,
        scratch_types=[
            pltpu.VMEM((rows_per + 1, c), jnp.float32),
            pltpu.VMEM((kpad,), jnp.int32),
            pltpu.VMEM((kpad,), jnp.int32),
            pltpu.VMEM((kpad,), jnp.int32),
            pltpu.VMEM((kpad,), jnp.float32),
            pltpu.SemaphoreType.DMA,
            pltpu.SemaphoreType.DMA,
        ],
        compiler_params=pltpu.CompilerParams(needs_layout_passes=False,
                                             use_tc_tiling_on_sc=True),
    )
    def body(x_hbm, xflat_hbm, srcs_hbm, offr_hbm, offc_hbm, out_hbm,
             chunk_v, src_v, offr_v, offc_v, val_v, sem_c, sem_g):
        wid = lax.axis_index("s") * _NC + lax.axis_index("c")
        row0 = wid * rows_per
        kq = kpad // _NSTREAM
        with jax.named_scope("sc_issue"):
            cp_in = pltpu.make_async_copy(
                x_hbm.at[pl.ds(row0, rows_per), :],
                chunk_v.at[pl.ds(0, rows_per), :], sem_c)
            cp_in.start()
            pltpu.sync_copy(srcs_hbm.at[wid], src_v)
            gats = []
            for q in range(_NSTREAM):
                gats.append(pltpu.make_async_copy(
                    xflat_hbm.at[src_v.at[pl.ds(q * kq, kq)]],
                    val_v.at[pl.ds(q * kq, kq)], sem_g))
                gats[-1].start()
            pltpu.sync_copy(offr_hbm.at[wid], offr_v)
            pltpu.sync_copy(offc_hbm.at[wid], offc_v)
        with jax.named_scope("sc_wait_dense"):
            cp_in.wait()
        with jax.named_scope("sc_wait_gather"):
            for g in gats:
                g.wait()

        with jax.named_scope("sc_fix"):
            @plsc.parallel_loop(0, kpad, step=_L, unroll=8)
            def fix(i):
                offr = offr_v[pl.ds(i, _L)]
                offc = offc_v[pl.ds(i, _L)]
                vals = val_v[pl.ds(i, _L)]
                plsc.store_scatter(chunk_v, [offr, offc], vals)

        with jax.named_scope("sc_out"):
            pltpu.sync_copy(chunk_v.at[pl.ds(0, rows_per), :],
                            out_hbm.at[pl.ds(row0, rows_per), :])

    s_const = jnp.asarray(src_all)
    r_const = jnp.asarray(offr_all)
    c_const = jnp.asarray(offc_all)

    def run(x, xflat):
        return body(x, xflat, s_const, r_const, c_const)

    return run


def kernel(x):
    b, c = x.shape
    return _build(b, c)(x, x.reshape(-1))


# speedup vs baseline: 1.8529x; 1.8529x over previous
import functools

import numpy as np
import jax
import jax.numpy as jnp
from jax import lax
from jax.experimental import pallas as pl
from jax.experimental.pallas import tpu as pltpu
from jax.experimental.pallas import tpu_sc as plsc

_NC, _NS, _L = 2, 16, 16
_NW = _NC * _NS
_P = 0.15


def _threefry2x32(k0, k1, x0, x1):
    x0 = x0.astype(np.uint32).copy()
    x1 = x1.astype(np.uint32).copy()

    def rotl(v, r):
        return ((v << np.uint32(r)) | (v >> np.uint32(32 - r))).astype(np.uint32)

    ks0 = np.uint32(k0)
    ks1 = np.uint32(k1)
    ks2 = np.uint32(ks0 ^ ks1 ^ np.uint32(0x1BD11BDA))
    ks = (ks0, ks1, ks2)
    x0 = (x0 + ks0).astype(np.uint32)
    x1 = (x1 + ks1).astype(np.uint32)
    r1 = (13, 15, 26, 6)
    r2 = (17, 29, 16, 24)
    for r in range(5):
        for rot in (r1 if r % 2 == 0 else r2):
            x0 = (x0 + x1).astype(np.uint32)
            x1 = rotl(x1, rot) ^ x0
        x0 = (x0 + ks[(r + 1) % 3]).astype(np.uint32)
        x1 = (x1 + ks[(r + 2) % 3] + np.uint32(r + 1)).astype(np.uint32)
    return x0, x1


def _np_bits(k0, k1, n):
    y0, y1 = _threefry2x32(k0, k1, np.zeros(n, np.uint32),
                           np.arange(n, dtype=np.uint32))
    return y0 ^ y1


def _np_split(k0, k1):
    y0, y1 = _threefry2x32(k0, k1, np.zeros(2, np.uint32),
                           np.arange(2, dtype=np.uint32))
    return (y0[0], y1[0]), (y0[1], y1[1])


def _np_uniform(k0, k1, m):
    bits = _np_bits(k0, k1, m)
    fb = (bits >> np.uint32(9)) | np.uint32(0x3F800000)
    return fb.view(np.float32) - np.float32(1.0)


@functools.lru_cache(maxsize=None)
def _swap_plan(b, c):
    n = b * c
    (k10, k11), (k20, k21) = _np_split(0, 42)
    u1 = _np_uniform(k10, k11, n).reshape(b, c)
    u2 = _np_uniform(k20, k21, n).reshape(b, c)
    mask = u1 > np.float32(1.0 - _P)
    rows = np.floor(u2 * np.float32(b)).astype(np.int32)
    delta = (rows.astype(np.int64) * mask.astype(np.int64) * c).reshape(-1)
    src = np.arange(n, dtype=np.int64) + delta
    src = np.where(src >= n, src - n, src).astype(np.int32)

    rows_per = b // _NW
    cs = rows_per * c
    swapped = np.nonzero(src != np.arange(n, dtype=np.int32))[0].astype(np.int64)
    dw = (swapped // cs).astype(np.int32)
    sw = (src[swapped] // cs).astype(np.int32)

    cnt = np.zeros((_NW, _NW), dtype=np.int64)
    np.add.at(cnt, (sw, dw), 1)
    padq = int(-(-cnt.max() // 8) * 8)
    k1 = _NW * padq

    kmaxd = np.bincount(dw, minlength=_NW).max()
    kpad = int(-(-kmaxd // _L) * _L)

    srcrc_all = np.zeros((_NW, k1), dtype=np.int32)
    pos_all = np.zeros((_NW, kpad), dtype=np.int32)
    drc_all = np.empty((_NW, kpad), dtype=np.int32)
    drc_all[:] = (rows_per << 7) + (np.arange(kpad, dtype=np.int32) % _L)

    fill_d = np.zeros(_NW, dtype=np.int64)
    rank_in_pair = np.zeros((_NW, _NW), dtype=np.int64)
    for ws in range(_NW):
        sel = np.nonzero(sw == ws)[0]
        ents = swapped[sel]
        esrc = src[ents]
        edw = dw[sel]
        sloc = esrc - np.int64(ws) * cs
        srcrc = ((sloc // c) << 7) + (sloc % c)
        ranks = np.empty(len(sel), dtype=np.int64)
        for i, wdv in enumerate(edw):
            ranks[i] = rank_in_pair[ws, wdv]
            rank_in_pair[ws, wdv] += 1
        slots = edw.astype(np.int64) * padq + ranks
        srcrc_all[ws, slots] = srcrc.astype(np.int32)
        for i, wdv in enumerate(edw):
            j = fill_d[wdv]
            pos_all[wdv, j] = ws * padq + ranks[i]
            dl = ents[i] - np.int64(wdv) * cs
            drc_all[wdv, j] = ((dl // c) << 7) + (dl % c)
            fill_d[wdv] += 1
    return rows_per, padq, kpad, srcrc_all, pos_all, drc_all


@functools.lru_cache(maxsize=None)
def _build(b, c):
    rows_per, padq, kpad, srcrc_all, pos_all, drc_all = _swap_plan(b, c)
    k1 = _NW * padq
    nmid = _NW * k1
    mesh = plsc.VectorSubcoreMesh(core_axis_name="c", subcore_axis_name="s",
                                  num_cores=_NC, num_subcores=_NS)

    @functools.partial(
        pl.kernel,
        out_type=jax.ShapeDtypeStruct((nmid,), jnp.float32),
        mesh=mesh,
        scratch_types=[
            pltpu.VMEM((rows_per, c), jnp.float32),
            pltpu.VMEM((k1,), jnp.int32),
            pltpu.VMEM((k1,), jnp.float32),
            pltpu.SemaphoreType.DMA,
        ],
        compiler_params=pltpu.CompilerParams(needs_layout_passes=False),
    )
    def extract(x_hbm, srcrc_hbm, mid_hbm, chunk_v, rc_v, seg_v, sem_c):
        ws = lax.axis_index("s") * _NC + lax.axis_index("c")
        with jax.named_scope("e_issue"):
            cp_in = pltpu.make_async_copy(
                x_hbm.at[pl.ds(ws * rows_per, rows_per), :], chunk_v, sem_c)
            cp_in.start()
            pltpu.sync_copy(srcrc_hbm.at[ws], rc_v)
        with jax.named_scope("e_wait"):
            cp_in.wait()

        with jax.named_scope("e_gather"):
            @plsc.parallel_loop(0, k1, step=_L, unroll=8)
            def ext(i):
                rc = rc_v[pl.ds(i, _L)]
                r = lax.shift_right_logical(rc, 7)
                col = lax.bitwise_and(rc, 127)
                seg_v[pl.ds(i, _L)] = plsc.load_gather(chunk_v, [r, col])

        with jax.named_scope("e_out"):
            pltpu.sync_copy(seg_v, mid_hbm.at[pl.ds(ws * k1, k1)])

    @functools.partial(
        pl.kernel,
        out_type=jax.ShapeDtypeStruct((b, c), jnp.float32),
        mesh=mesh,
        scratch_types=[
            pltpu.VMEM((rows_per + 1, c), jnp.float32),
            pltpu.VMEM((k1,), jnp.float32),
            pltpu.VMEM((kpad,), jnp.int32),
            pltpu.VMEM((kpad,), jnp.int32),
            pltpu.SemaphoreType.DMA,
            pltpu.SemaphoreType.DMA,
        ],
        compiler_params=pltpu.CompilerParams(needs_layout_passes=False),
    )
    def patch(x_hbm, mid_hbm, pos_hbm, drc_hbm, out_hbm,
              chunk_v, inc_v, pos_v, drc_v, sem_c, sem_m):
        wd = lax.axis_index("s") * _NC + lax.axis_index("c")
        row0 = wd * rows_per
        with jax.named_scope("p_issue"):
            cp_in = pltpu.make_async_copy(
                x_hbm.at[pl.ds(row0, rows_per), :],
                chunk_v.at[pl.ds(0, rows_per), :], sem_c)
            cp_in.start()
            incs = []
            for ws in range(_NW):
                incs.append(pltpu.make_async_copy(
                    mid_hbm.at[pl.ds(ws * k1 + wd * padq, padq)],
                    inc_v.at[pl.ds(ws * padq, padq)], sem_m))
                incs[-1].start()
            pltpu.sync_copy(pos_hbm.at[wd], pos_v)
            pltpu.sync_copy(drc_hbm.at[wd], drc_v)
        with jax.named_scope("p_wait"):
            for g in incs:
                g.wait()
            cp_in.wait()

        with jax.named_scope("p_fix"):
            @plsc.parallel_loop(0, kpad, step=_L, unroll=8)
            def fix(i):
                pos = pos_v[pl.ds(i, _L)]
                rc = drc_v[pl.ds(i, _L)]
                r = lax.shift_right_logical(rc, 7)
                col = lax.bitwise_and(rc, 127)
                vals = plsc.load_gather(inc_v, [pos])
                plsc.store_scatter(chunk_v, [r, col], vals)

        with jax.named_scope("p_out"):
            pltpu.sync_copy(chunk_v.at[pl.ds(0, rows_per), :],
                            out_hbm.at[pl.ds(row0, rows_per), :])

    rc_const = jnp.asarray(srcrc_all)
    pos_const = jnp.asarray(pos_all)
    drc_const = jnp.asarray(drc_all)

    def run(x):
        mid = extract(x, rc_const)
        return patch(x, mid, pos_const, drc_const)

    return run


def kernel(x):
    b, c = x.shape
    return _build(b, c)(x)


# speedup vs baseline: 1.8632x; 1.0056x over previous
import functools

import numpy as np
import jax
import jax.numpy as jnp
from jax import lax
from jax.experimental import pallas as pl
from jax.experimental.pallas import tpu as pltpu
from jax.experimental.pallas import tpu_sc as plsc

_NC, _NS, _L = 2, 16, 16
_NW = _NC * _NS
_P = 0.15


def _threefry2x32(k0, k1, x0, x1):
    x0 = x0.astype(np.uint32).copy()
    x1 = x1.astype(np.uint32).copy()

    def rotl(v, r):
        return ((v << np.uint32(r)) | (v >> np.uint32(32 - r))).astype(np.uint32)

    ks0 = np.uint32(k0)
    ks1 = np.uint32(k1)
    ks2 = np.uint32(ks0 ^ ks1 ^ np.uint32(0x1BD11BDA))
    ks = (ks0, ks1, ks2)
    x0 = (x0 + ks0).astype(np.uint32)
    x1 = (x1 + ks1).astype(np.uint32)
    r1 = (13, 15, 26, 6)
    r2 = (17, 29, 16, 24)
    for r in range(5):
        for rot in (r1 if r % 2 == 0 else r2):
            x0 = (x0 + x1).astype(np.uint32)
            x1 = rotl(x1, rot) ^ x0
        x0 = (x0 + ks[(r + 1) % 3]).astype(np.uint32)
        x1 = (x1 + ks[(r + 2) % 3] + np.uint32(r + 1)).astype(np.uint32)
    return x0, x1


def _np_bits(k0, k1, n):
    y0, y1 = _threefry2x32(k0, k1, np.zeros(n, np.uint32),
                           np.arange(n, dtype=np.uint32))
    return y0 ^ y1


def _np_split(k0, k1):
    y0, y1 = _threefry2x32(k0, k1, np.zeros(2, np.uint32),
                           np.arange(2, dtype=np.uint32))
    return (y0[0], y1[0]), (y0[1], y1[1])


def _np_uniform(k0, k1, m):
    bits = _np_bits(k0, k1, m)
    fb = (bits >> np.uint32(9)) | np.uint32(0x3F800000)
    return fb.view(np.float32) - np.float32(1.0)


@functools.lru_cache(maxsize=None)
def _swap_plan(b, c):
    n = b * c
    (k10, k11), (k20, k21) = _np_split(0, 42)
    u1 = _np_uniform(k10, k11, n).reshape(b, c)
    u2 = _np_uniform(k20, k21, n).reshape(b, c)
    mask = u1 > np.float32(1.0 - _P)
    rows = np.floor(u2 * np.float32(b)).astype(np.int32)
    delta = (rows.astype(np.int64) * mask.astype(np.int64) * c).reshape(-1)
    src = np.arange(n, dtype=np.int64) + delta
    src = np.where(src >= n, src - n, src).astype(np.int32)

    rows_per = b // _NW
    cs = rows_per * c
    swapped = np.nonzero(src != np.arange(n, dtype=np.int32))[0].astype(np.int64)
    dw = (swapped // cs).astype(np.int32)
    sw = (src[swapped] // cs).astype(np.int32)

    cnt = np.zeros((_NW, _NW), dtype=np.int64)
    np.add.at(cnt, (sw, dw), 1)
    padq = int(-(-cnt.max() // 8) * 8)
    k1 = _NW * padq

    kmaxd = np.bincount(dw, minlength=_NW).max()
    kpad = int(-(-kmaxd // _L) * _L)

    srcrc_all = np.zeros((_NW, k1), dtype=np.int32)
    pos_all = np.zeros((_NW, kpad), dtype=np.int32)
    drc_all = np.empty((_NW, kpad), dtype=np.int32)
    drc_all[:] = (rows_per << 7) + (np.arange(kpad, dtype=np.int32) % _L)

    fill_d = np.zeros(_NW, dtype=np.int64)
    rank_in_pair = np.zeros((_NW, _NW), dtype=np.int64)
    for ws in range(_NW):
        sel = np.nonzero(sw == ws)[0]
        ents = swapped[sel]
        esrc = src[ents]
        edw = dw[sel]
        sloc = esrc - np.int64(ws) * cs
        srcrc = ((sloc // c) << 7) + (sloc % c)
        ranks = np.empty(len(sel), dtype=np.int64)
        for i, wdv in enumerate(edw):
            ranks[i] = rank_in_pair[ws, wdv]
            rank_in_pair[ws, wdv] += 1
        slots = edw.astype(np.int64) * padq + ranks
        srcrc_all[ws, slots] = srcrc.astype(np.int32)
        for i, wdv in enumerate(edw):
            j = fill_d[wdv]
            pos_all[wdv, j] = ws * padq + ranks[i]
            dl = ents[i] - np.int64(wdv) * cs
            drc_all[wdv, j] = ((dl // c) << 7) + (dl % c)
            fill_d[wdv] += 1
    return rows_per, padq, kpad, srcrc_all, pos_all, drc_all


@functools.lru_cache(maxsize=None)
def _build(b, c):
    rows_per, padq, kpad, srcrc_all, pos_all, drc_all = _swap_plan(b, c)
    k1 = _NW * padq
    nmid = _NW * k1
    mesh = plsc.VectorSubcoreMesh(core_axis_name="c", subcore_axis_name="s",
                                  num_cores=_NC, num_subcores=_NS)

    @functools.partial(
        pl.kernel,
        out_type=jax.ShapeDtypeStruct((nmid,), jnp.float32),
        mesh=mesh,
        scratch_types=[
            pltpu.VMEM((rows_per, c), jnp.float32),
            pltpu.VMEM((k1,), jnp.int32),
            pltpu.VMEM((k1,), jnp.float32),
            pltpu.SemaphoreType.DMA,
        ],
        compiler_params=pltpu.CompilerParams(needs_layout_passes=False),
    )
    def extract(x_hbm, srcrc_hbm, mid_hbm, chunk_v, rc_v, seg_v, sem_c):
        ws = lax.axis_index("s") * _NC + lax.axis_index("c")
        with jax.named_scope("e_issue"):
            cp_in = pltpu.make_async_copy(
                x_hbm.at[pl.ds(ws * rows_per, rows_per), :], chunk_v, sem_c)
            cp_in.start()
            pltpu.sync_copy(srcrc_hbm.at[ws], rc_v)
        with jax.named_scope("e_wait"):
            cp_in.wait()

        with jax.named_scope("e_gather"):
            @plsc.parallel_loop(0, k1, step=_L, unroll=8)
            def ext(i):
                rc = rc_v[pl.ds(i, _L)]
                r = lax.shift_right_logical(rc, 7)
                col = lax.bitwise_and(rc, 127)
                seg_v[pl.ds(i, _L)] = plsc.load_gather(chunk_v, [r, col])

        with jax.named_scope("e_out"):
            outs = []
            for wd in range(_NW):
                outs.append(pltpu.make_async_copy(
                    seg_v.at[pl.ds(wd * padq, padq)],
                    mid_hbm.at[pl.ds((wd * _NW + ws) * padq, padq)], sem_c))
                outs[-1].start()
            for g in outs:
                g.wait()

    @functools.partial(
        pl.kernel,
        out_type=jax.ShapeDtypeStruct((b, c), jnp.float32),
        mesh=mesh,
        scratch_types=[
            pltpu.VMEM((rows_per + 1, c), jnp.float32),
            pltpu.VMEM((k1,), jnp.float32),
            pltpu.VMEM((2 * kpad,), jnp.int32),
            pltpu.SemaphoreType.DMA,
            pltpu.SemaphoreType.DMA,
        ],
        compiler_params=pltpu.CompilerParams(needs_layout_passes=False),
    )
    def patch(x_hbm, mid_hbm, pd_hbm, out_hbm,
              chunk_v, inc_v, pd_v, sem_c, sem_m):
        wd = lax.axis_index("s") * _NC + lax.axis_index("c")
        row0 = wd * rows_per
        with jax.named_scope("p_issue"):
            cp_in = pltpu.make_async_copy(
                x_hbm.at[pl.ds(row0, rows_per), :],
                chunk_v.at[pl.ds(0, rows_per), :], sem_c)
            cp_in.start()
            inc = pltpu.make_async_copy(
                mid_hbm.at[pl.ds(wd * k1, k1)], inc_v, sem_m)
            inc.start()
            pltpu.sync_copy(pd_hbm.at[wd], pd_v)
        with jax.named_scope("p_wait"):
            inc.wait()
            cp_in.wait()

        with jax.named_scope("p_fix"):
            @plsc.parallel_loop(0, kpad, step=_L, unroll=8)
            def fix(i):
                pos = pd_v[pl.ds(i, _L)]
                rc = pd_v[pl.ds(kpad + i, _L)]
                r = lax.shift_right_logical(rc, 7)
                col = lax.bitwise_and(rc, 127)
                vals = plsc.load_gather(inc_v, [pos])
                plsc.store_scatter(chunk_v, [r, col], vals)

        with jax.named_scope("p_out"):
            pltpu.sync_copy(chunk_v.at[pl.ds(0, rows_per), :],
                            out_hbm.at[pl.ds(row0, rows_per), :])

    rc_const = jnp.asarray(srcrc_all)
    pd_const = jnp.asarray(np.concatenate([pos_all, drc_all], axis=1))

    def run(x):
        mid = extract(x, rc_const)
        return patch(x, mid, pd_const)

    return run


def kernel(x):
    b, c = x.shape
    return _build(b, c)(x)
